# baseline (speedup 1.0000x reference)
import functools

import jax
import jax.numpy as jnp
from jax import lax
from jax.experimental import pallas as pl
from jax.experimental.pallas import tpu as pltpu
from jax.experimental.pallas import tpu_sc as plsc

CYCLE = 168
LEN = 336
D = 128
B = 1024

NC = 2
NS = 16
NW = NC * NS
BPW = B // NW


def _sc_cycle_gather(index, data):
  mesh = plsc.VectorSubcoreMesh(core_axis_name="c", subcore_axis_name="s")

  @functools.partial(
      pl.kernel,
      out_type=jax.ShapeDtypeStruct((B * LEN, D), jnp.float32),
      mesh=mesh,
      scratch_types=[
          pltpu.VMEM((BPW + 16,), jnp.int32),
          pltpu.VMEM_SHARED((3 * CYCLE, D), jnp.float32),
          pltpu.SemaphoreType.DMA,
          pltpu.SemaphoreType.DMA,
      ],
  )
  def k(index_hbm, data_hbm, out_hbm, sidx_v, d3_sh, tsem, wsem):
    sid = lax.axis_index("s")
    wid = sid * NC + lax.axis_index("c")
    base = wid * BPW

    @pl.when(sid == 0)
    def _():
      pltpu.async_copy(data_hbm, d3_sh.at[pl.ds(0, CYCLE)], tsem)
      pltpu.async_copy(data_hbm, d3_sh.at[pl.ds(CYCLE, CYCLE)], tsem)
      pltpu.async_copy(data_hbm, d3_sh.at[pl.ds(2 * CYCLE, CYCLE)], tsem)

    pltpu.sync_copy(index_hbm.at[pl.ds(base, BPW)], sidx_v.at[pl.ds(0, BPW)])

    @pl.when(sid == 0)
    def _():
      pltpu.make_async_copy(data_hbm, d3_sh.at[pl.ds(0, CYCLE)], tsem).wait()
      pltpu.make_async_copy(data_hbm, d3_sh.at[pl.ds(0, CYCLE)], tsem).wait()
      pltpu.make_async_copy(data_hbm, d3_sh.at[pl.ds(0, CYCLE)], tsem).wait()

    plsc.subcore_barrier()

    def body(s, carry):
      r = sidx_v[pl.ds(s, 16)][0]
      pltpu.async_copy(
          d3_sh.at[pl.ds(r, LEN)], out_hbm.at[pl.ds((base + s) * LEN, LEN)],
          wsem)
      return carry

    lax.fori_loop(0, BPW, body, 0)

    def drain(s, carry):
      pltpu.make_async_copy(
          d3_sh.at[pl.ds(0, LEN)], out_hbm.at[pl.ds(base * LEN, LEN)],
          wsem).wait()
      return carry

    lax.fori_loop(0, BPW, drain, 0)

  return k(index, data)


def kernel(index, length, data):
  del length
  out = _sc_cycle_gather(index.astype(jnp.int32), data)
  return out.reshape(B, LEN, D)


# speedup vs baseline: 1.3192x; 1.3192x over previous
import functools

import jax
import jax.numpy as jnp
from jax import lax
from jax.experimental import pallas as pl
from jax.experimental.pallas import tpu as pltpu
from jax.experimental.pallas import tpu_sc as plsc

CYCLE = 168
LEN = 336
D = 128
B = 1024

NC = 2
NS = 16
NW = NC * NS
BPW = B // NW


def _sc_cycle_gather(index, data):
  mesh = plsc.VectorSubcoreMesh(core_axis_name="c", subcore_axis_name="s")

  @functools.partial(
      pl.kernel,
      out_type=jax.ShapeDtypeStruct((B * LEN, D), jnp.float32),
      mesh=mesh,
      scratch_types=[
          pltpu.VMEM((BPW + 16,), jnp.int32),
          pltpu.VMEM((2 * CYCLE, D), jnp.float32),
          pltpu.SemaphoreType.DMA,
          pltpu.SemaphoreType.DMA,
      ],
  )
  def k(index_hbm, data_hbm, out_hbm, sidx_v, d2_v, tsem, wsem):
    wid = lax.axis_index("s") * NC + lax.axis_index("c")
    base = wid * BPW

    cp0 = pltpu.async_copy(data_hbm, d2_v.at[pl.ds(0, CYCLE)], tsem)
    cp1 = pltpu.async_copy(data_hbm, d2_v.at[pl.ds(CYCLE, CYCLE)], tsem)
    pltpu.sync_copy(index_hbm.at[pl.ds(base, BPW)], sidx_v.at[pl.ds(0, BPW)])
    cp0.wait()
    cp1.wait()

    def body(s, carry):
      r = sidx_v[pl.ds(s, 16)][0]
      src = d2_v.at[pl.ds(r, CYCLE)]
      pltpu.async_copy(
          src, out_hbm.at[pl.ds((base + s) * LEN, CYCLE)], wsem)
      pltpu.async_copy(
          src, out_hbm.at[pl.ds((base + s) * LEN + CYCLE, CYCLE)], wsem)
      return carry

    lax.fori_loop(0, BPW, body, 0)

    def drain(s, carry):
      pltpu.make_async_copy(
          d2_v.at[pl.ds(0, CYCLE)], out_hbm.at[pl.ds(base * LEN, CYCLE)],
          wsem).wait()
      return carry

    lax.fori_loop(0, 2 * BPW, drain, 0)

  return k(index, data)


def kernel(index, length, data):
  del length
  out = _sc_cycle_gather(index.astype(jnp.int32), data)
  return out.reshape(B, LEN, D)
